# baseline (speedup 1.0000x reference)
import functools

import jax
import jax.numpy as jnp
from jax import lax
from jax.experimental import pallas as pl
from jax.experimental.pallas import tpu as pltpu
from jax.experimental.pallas import tpu_sc as plsc

_R, _C = 16384, 200
_D = 32
_NC, _NS = 2, 16
_NW = _NC * _NS
_ROWS_W = _R // _NW
_BLK = 8
_NCHUNK = _ROWS_W // _BLK
_SPLITS = ((0, 128), (128, 72))


def _relu_body(t_ref, o_ref):
    o_ref[...] = jnp.maximum(t_ref[...], 0.0)


def _relu_table(table):
    return pl.pallas_call(
        _relu_body,
        out_shape=jax.ShapeDtypeStruct(table.shape, table.dtype),
    )(table)


@functools.partial(
    pl.kernel,
    mesh=plsc.VectorSubcoreMesh(core_axis_name="c", subcore_axis_name="s"),
    compiler_params=pltpu.CompilerParams(use_tc_tiling_on_sc=False),
    out_type=jax.ShapeDtypeStruct((_R, _C, _D), jnp.float32),
    scratch_types=[
        pltpu.VMEM((_BLK, _C), jnp.int32),
        pltpu.VMEM((_BLK, _C, _D), jnp.float32),
        pltpu.SemaphoreType.DMA,
    ],
)
def _sc_gather(table_hbm, idx_hbm, out_hbm, idx_v, rows_v, sem):
    wid = lax.axis_index("s") * _NC + lax.axis_index("c")
    base_row = wid * _ROWS_W

    def body(i, carry):
        r0 = base_row + i * _BLK
        pltpu.sync_copy(idx_hbm.at[pl.ds(r0, _BLK)], idx_v)
        copies = [
            pltpu.async_copy(
                table_hbm.at[idx_v.at[r, pl.ds(o, n)]],
                rows_v.at[r, pl.ds(o, n)],
                sem,
            )
            for r in range(_BLK)
            for (o, n) in _SPLITS
        ]
        for c in copies:
            c.wait()
        pltpu.sync_copy(rows_v, out_hbm.at[pl.ds(r0, _BLK)])
        return carry

    lax.fori_loop(0, _NCHUNK, body, 0)


def kernel(x, kernel):
    idx = x.astype(jnp.int32)
    table = _relu_table(kernel)
    return _sc_gather(table, idx)


# speedup vs baseline: 1.1917x; 1.1917x over previous
import functools

import jax
import jax.numpy as jnp
from jax import lax
from jax.experimental import pallas as pl
from jax.experimental.pallas import tpu as pltpu
from jax.experimental.pallas import tpu_sc as plsc

_I, _J = 16384, 200
_V, _D = 10000, 32
_NC, _NS = 2, 16
_CH = 4096
_NCH = _I // _CH
_ROWS = _CH // 128


@functools.partial(
    pl.kernel,
    mesh=plsc.VectorSubcoreMesh(core_axis_name="c", subcore_axis_name="s"),
    compiler_params=pltpu.CompilerParams(
        use_tc_tiling_on_sc=False, needs_layout_passes=False
    ),
    out_type=jax.ShapeDtypeStruct((_J, _D // 8, _I // 128, 8, 128), jnp.float32),
    scratch_types=[
        pltpu.VMEM((_V,), jnp.float32),
        pltpu.VMEM((_CH,), jnp.int32),
        pltpu.VMEM((_ROWS, 128), jnp.float32),
        pltpu.SemaphoreType.DMA,
    ],
)
def _sc_lookup(tableT_hbm, idxT_hbm, out_hbm, tbl_v, idx_v, out_v, sem):
    k = lax.axis_index("s") * _NC + lax.axis_index("c")
    ktile, ksub = k // 8, k % 8

    pltpu.sync_copy(tableT_hbm.at[k], tbl_v)

    def relu_body(i, carry):
        tbl_v[pl.ds(i * 16, 16)] = jnp.maximum(tbl_v[pl.ds(i * 16, 16)], 0.0)
        return carry

    lax.fori_loop(0, _V // 16, relu_body, 0)

    def jbody(j, carry):
        for ci in range(_NCH):
            pltpu.sync_copy(idxT_hbm.at[j, pl.ds(ci * _CH, _CH)], idx_v)

            def ibody(i, c2):
                base = i * 128
                for u in range(8):
                    vec = idx_v[pl.ds(base + u * 16, 16)]
                    out_v[i, pl.ds(u * 16, 16)] = plsc.load_gather(tbl_v, [vec])
                return c2

            lax.fori_loop(0, _ROWS, ibody, 0)
            pltpu.sync_copy(out_v, out_hbm.at[j, ktile, pl.ds(ci * _ROWS, _ROWS), ksub])
        return carry

    lax.fori_loop(0, _J, jbody, 0)


def kernel(x, kernel):
    idxT = jnp.transpose(x.astype(jnp.int32))
    tableT = jnp.transpose(kernel)
    out5 = _sc_lookup(tableT, idxT)
    return jnp.transpose(out5, (2, 4, 0, 1, 3)).reshape(_I, _J, _D)


# speedup vs baseline: 3.3342x; 2.7979x over previous
import functools

import jax
import jax.numpy as jnp
from jax import lax
from jax.experimental import pallas as pl
from jax.experimental.pallas import tpu as pltpu
from jax.experimental.pallas import tpu_sc as plsc

_I, _J = 16384, 200
_V, _D = 10000, 32
_NC, _NS = 2, 16
_CH = 4096
_NCH = _I // _CH
_ROWS = _CH // 128
_NCHUNK = _J * _NCH


@functools.partial(
    pl.kernel,
    mesh=plsc.VectorSubcoreMesh(core_axis_name="c", subcore_axis_name="s"),
    compiler_params=pltpu.CompilerParams(
        use_tc_tiling_on_sc=False, needs_layout_passes=False
    ),
    out_type=jax.ShapeDtypeStruct((_J, _D // 8, _I // 128, 8, 128), jnp.float32),
    scratch_types=[
        pltpu.VMEM((_V,), jnp.float32),
        pltpu.VMEM((2, _CH), jnp.int32),
        pltpu.VMEM((2, _ROWS, 128), jnp.float32),
        pltpu.SemaphoreType.DMA,
        pltpu.SemaphoreType.DMA,
        pltpu.SemaphoreType.DMA,
        pltpu.SemaphoreType.DMA,
    ],
)
def _sc_lookup(tableT_hbm, idxT_hbm, out_hbm, tbl_v, idx_v, out_v,
               sem_i0, sem_i1, sem_o0, sem_o1):
    sems_i = (sem_i0, sem_i1)
    sems_o = (sem_o0, sem_o1)
    k = lax.axis_index("s") * _NC + lax.axis_index("c")
    ktile, ksub = k // 8, k % 8

    pltpu.sync_copy(tableT_hbm.at[k], tbl_v)

    def relu_body(i, carry):
        tbl_v[pl.ds(i * 16, 16)] = jnp.maximum(tbl_v[pl.ds(i * 16, 16)], 0.0)
        return carry

    lax.fori_loop(0, _V // 16, relu_body, 0)

    pltpu.async_copy(idxT_hbm.at[0, pl.ds(0, _CH)], idx_v.at[0], sems_i[0])
    pltpu.async_copy(idxT_hbm.at[0, pl.ds(_CH, _CH)], idx_v.at[1], sems_i[1])

    def body(c2, carry):
        for b in range(2):
            c = c2 * 2 + b
            j, ci = c // _NCH, c % _NCH
            pltpu.make_async_copy(
                idxT_hbm.at[0, pl.ds(0, _CH)], idx_v.at[b], sems_i[b]
            ).wait()

            @pl.when(c2 > 0)
            def _drain():
                pltpu.make_async_copy(
                    out_v.at[b], out_hbm.at[0, 0, pl.ds(0, _ROWS), 0], sems_o[b]
                ).wait()

            @plsc.parallel_loop(0, _ROWS, 1, unroll=4)
            def ibody(i):
                base = i * 128
                for u in range(8):
                    vec = idx_v[b, pl.ds(base + u * 16, 16)]
                    out_v[b, i, pl.ds(u * 16, 16)] = plsc.load_gather(tbl_v, [vec])

            pltpu.async_copy(
                out_v.at[b],
                out_hbm.at[j, ktile, pl.ds(ci * _ROWS, _ROWS), ksub],
                sems_o[b],
            )

            cn = c + 2
            jn, cin = cn // _NCH, cn % _NCH

            @pl.when(cn < _NCHUNK)
            def _prefetch():
                pltpu.async_copy(
                    idxT_hbm.at[jn, pl.ds(cin * _CH, _CH)], idx_v.at[b], sems_i[b]
                )

        return carry

    lax.fori_loop(0, _NCHUNK // 2, body, 0)

    for b in range(2):
        pltpu.make_async_copy(
            out_v.at[b], out_hbm.at[0, 0, pl.ds(0, _ROWS), 0], sems_o[b]
        ).wait()


def kernel(x, kernel):
    idxT = jnp.transpose(x.astype(jnp.int32))
    tableT = jnp.transpose(kernel)
    out5 = _sc_lookup(tableT, idxT)
    return jnp.transpose(out5, (2, 4, 0, 1, 3)).reshape(_I, _J, _D)


# speedup vs baseline: 4.3251x; 1.2972x over previous
import functools

import jax
import jax.numpy as jnp
from jax import lax
from jax.experimental import pallas as pl
from jax.experimental.pallas import tpu as pltpu
from jax.experimental.pallas import tpu_sc as plsc

_I, _J = 16384, 200
_V, _D = 10000, 32
_NC, _NS = 2, 16
_CH = 8192
_NCH = _I // _CH
_ROWS = _CH // 128
_NCHUNK = _J * _NCH


@functools.partial(
    pl.kernel,
    mesh=plsc.VectorSubcoreMesh(core_axis_name="c", subcore_axis_name="s"),
    compiler_params=pltpu.CompilerParams(
        use_tc_tiling_on_sc=False, needs_layout_passes=False
    ),
    out_type=jax.ShapeDtypeStruct((_J, _D // 8, _I // 128, 8, 128), jnp.float32),
    scratch_types=[
        pltpu.VMEM((_V,), jnp.float32),
        pltpu.VMEM((2, _CH), jnp.int32),
        pltpu.VMEM((2, _ROWS, 128), jnp.float32),
        pltpu.SemaphoreType.DMA,
        pltpu.SemaphoreType.DMA,
        pltpu.SemaphoreType.DMA,
        pltpu.SemaphoreType.DMA,
    ],
)
def _sc_lookup(tableT_hbm, idxT_hbm, out_hbm, tbl_v, idx_v, out_v,
               sem_i0, sem_i1, sem_o0, sem_o1):
    sems_i = (sem_i0, sem_i1)
    sems_o = (sem_o0, sem_o1)
    k = lax.axis_index("s") * _NC + lax.axis_index("c")
    ktile, ksub = k // 8, k % 8

    pltpu.sync_copy(tableT_hbm.at[k], tbl_v)

    def relu_body(i, carry):
        tbl_v[pl.ds(i * 16, 16)] = jnp.maximum(tbl_v[pl.ds(i * 16, 16)], 0.0)
        return carry

    lax.fori_loop(0, _V // 16, relu_body, 0)

    pltpu.async_copy(idxT_hbm.at[0, pl.ds(0, _CH)], idx_v.at[0], sems_i[0])
    pltpu.async_copy(idxT_hbm.at[0, pl.ds(_CH, _CH)], idx_v.at[1], sems_i[1])

    def body(c2, carry):
        for b in range(2):
            c = c2 * 2 + b
            j, ci = c // _NCH, c % _NCH
            pltpu.make_async_copy(
                idxT_hbm.at[0, pl.ds(0, _CH)], idx_v.at[b], sems_i[b]
            ).wait()

            @pl.when(c2 > 0)
            def _drain():
                pltpu.make_async_copy(
                    out_v.at[b], out_hbm.at[0, 0, pl.ds(0, _ROWS), 0], sems_o[b]
                ).wait()

            @plsc.parallel_loop(0, _ROWS, 1, unroll=8)
            def ibody(i):
                base = i * 128
                for u in range(8):
                    vec = idx_v[b, pl.ds(base + u * 16, 16)]
                    out_v[b, i, pl.ds(u * 16, 16)] = plsc.load_gather(tbl_v, [vec])

            pltpu.async_copy(
                out_v.at[b],
                out_hbm.at[j, ktile, pl.ds(ci * _ROWS, _ROWS), ksub],
                sems_o[b],
            )

            cn = c + 2
            jn, cin = cn // _NCH, cn % _NCH

            @pl.when(cn < _NCHUNK)
            def _prefetch():
                pltpu.async_copy(
                    idxT_hbm.at[jn, pl.ds(cin * _CH, _CH)], idx_v.at[b], sems_i[b]
                )

        return carry

    lax.fori_loop(0, _NCHUNK // 2, body, 0)

    for b in range(2):
        pltpu.make_async_copy(
            out_v.at[b], out_hbm.at[0, 0, pl.ds(0, _ROWS), 0], sems_o[b]
        ).wait()


def kernel(x, kernel):
    idxT = jnp.transpose(x.astype(jnp.int32))
    tableT = jnp.transpose(kernel)
    out5 = _sc_lookup(tableT, idxT)
    return jnp.transpose(out5, (2, 4, 0, 1, 3)).reshape(_I, _J, _D)


# speedup vs baseline: 5.7441x; 1.3281x over previous
import functools

import jax
import jax.numpy as jnp
from jax import lax
from jax.experimental import pallas as pl
from jax.experimental.pallas import tpu as pltpu
from jax.experimental.pallas import tpu_sc as plsc

_I, _J = 16384, 200
_V, _D = 10000, 32
_NC, _NS = 2, 16
_NKT = _D // 8
_NE = 8
_CH = _I // _NE
_TI = _CH // 128


@functools.partial(
    pl.kernel,
    mesh=plsc.VectorSubcoreMesh(core_axis_name="c", subcore_axis_name="s"),
    compiler_params=pltpu.CompilerParams(
        use_tc_tiling_on_sc=False, needs_layout_passes=False
    ),
    out_type=jax.ShapeDtypeStruct((_J, _NKT, _I // 128, 8, 128), jnp.float32),
    scratch_types=(
        [pltpu.VMEM((_V,), jnp.float32) for _ in range(8)]
        + [
            pltpu.VMEM((2, _CH), jnp.int32),
            pltpu.VMEM((2, _TI, 8, 128), jnp.float32),
            pltpu.SemaphoreType.DMA,
            pltpu.SemaphoreType.DMA,
            pltpu.SemaphoreType.DMA,
            pltpu.SemaphoreType.DMA,
        ]
    ),
)
def _sc_lookup(tableT_hbm, idxT_hbm, out_hbm,
               t0, t1, t2, t3, t4, t5, t6, t7,
               idx_v, out_v, sem_i0, sem_i1, sem_o0, sem_o1):
    tbls = (t0, t1, t2, t3, t4, t5, t6, t7)
    sems_i = (sem_i0, sem_i1)
    sems_o = (sem_o0, sem_o1)
    w = lax.axis_index("s") * _NC + lax.axis_index("c")
    ktile = w % _NKT
    e = w // _NKT
    ibase = e * _CH

    for d in range(8):
        pltpu.sync_copy(tableT_hbm.at[ktile * 8 + d], tbls[d])

    def relu_body(i, carry):
        for d in range(8):
            tbls[d][pl.ds(i * 16, 16)] = jnp.maximum(tbls[d][pl.ds(i * 16, 16)], 0.0)
        return carry

    lax.fori_loop(0, _V // 16, relu_body, 0)

    pltpu.async_copy(idxT_hbm.at[0, pl.ds(ibase, _CH)], idx_v.at[0], sems_i[0])
    pltpu.async_copy(idxT_hbm.at[1, pl.ds(ibase, _CH)], idx_v.at[1], sems_i[1])

    def body(j2, carry):
        for b in range(2):
            j = j2 * 2 + b
            pltpu.make_async_copy(
                idxT_hbm.at[0, pl.ds(0, _CH)], idx_v.at[b], sems_i[b]
            ).wait()

            @pl.when(j2 > 0)
            def _drain():
                pltpu.make_async_copy(
                    out_v.at[b], out_hbm.at[0, 0, pl.ds(0, _TI)], sems_o[b]
                ).wait()

            @plsc.parallel_loop(0, _TI, 1, unroll=2)
            def ibody(r):
                base = r * 128
                for u in range(8):
                    vec = idx_v[b, pl.ds(base + u * 16, 16)]
                    for d in range(8):
                        out_v[b, r, d, pl.ds(u * 16, 16)] = plsc.load_gather(
                            tbls[d], [vec]
                        )

            pltpu.async_copy(
                out_v.at[b],
                out_hbm.at[j, ktile, pl.ds(e * _TI, _TI)],
                sems_o[b],
            )

            jn = j + 2

            @pl.when(jn < _J)
            def _prefetch():
                pltpu.async_copy(
                    idxT_hbm.at[jn, pl.ds(ibase, _CH)], idx_v.at[b], sems_i[b]
                )

        return carry

    lax.fori_loop(0, _J // 2, body, 0)

    for b in range(2):
        pltpu.make_async_copy(
            out_v.at[b], out_hbm.at[0, 0, pl.ds(0, _TI)], sems_o[b]
        ).wait()


def kernel(x, kernel):
    idxT = jnp.transpose(x.astype(jnp.int32))
    tableT = jnp.transpose(kernel)
    out5 = _sc_lookup(tableT, idxT)
    return jnp.transpose(out5, (2, 4, 0, 1, 3)).reshape(_I, _J, _D)
